# baseline (speedup 1.0000x reference)
import jax
import jax.numpy as jnp
from jax.experimental import pallas as pl
from jax.experimental.pallas import tpu as pltpu

f32 = jnp.float32
bf16 = jnp.bfloat16

S = 5


def _dot(a, b):
    return jax.lax.dot_general(a, b, (((1,), (0,)), ((), ())),
                               preferred_element_type=f32)


def kernel(x, adj, adj_high, W0, b0, W1, b1, W2, b2, W_out, b_out):
    N, F = x.shape
    H = W0.shape[1]
    C = W_out.shape[1]
    R = N // S
    BM1 = 80
    BM2 = 200
    nb1 = R // BM1
    nb2 = R // BM2

    x_bf = x.astype(bf16)
    W0b = W0.astype(bf16)
    W1x = W1[:F].astype(bf16)
    W1h = W1[F:].astype(bf16)
    Wc_x = jnp.concatenate([W2[:F], W_out[:F]], axis=1).astype(bf16)
    Wc_h0 = jnp.concatenate([W2[F:F + H], W_out[F:F + H]], axis=1).astype(bf16)
    Wc_h1 = jnp.concatenate([W2[F + H:], W_out[F + H:F + 2 * H]], axis=1).astype(bf16)
    Wo2 = W_out[F + 2 * H:].astype(bf16)
    b0r = b0.reshape(1, H).astype(f32)
    b1r = b1.reshape(1, H).astype(f32)
    b2r = b2.reshape(1, H).astype(f32)
    boutr = b_out.reshape(1, C).astype(f32)

    cparams = pltpu.CompilerParams(dimension_semantics=("arbitrary",))

    def stream_spec(bm, w, nb):
        return [pl.BlockSpec((bm, w), (lambda i, s=s: (i + s * nb, 0)))
                for s in range(S)]

    def local_spec(bm, w):
        return [pl.BlockSpec((bm, w), lambda i: (i, 0)) for _ in range(S)]

    def full_spec(r, w):
        return [pl.BlockSpec((r, w), lambda i: (0, 0)) for _ in range(S)]

    def one(shape):
        return pl.BlockSpec(shape, lambda i: (0, 0))

    def shapes(r, w, dt):
        return [jax.ShapeDtypeStruct((r, w), dt) for _ in range(S)]

    def b1_body(x_ref, w_ref, o_ref):
        o_ref[...] = _dot(x_ref[...], w_ref[...]).astype(bf16)

    B1 = pl.pallas_call(
        b1_body,
        grid=(N // 1000,),
        in_specs=[pl.BlockSpec((1000, F), lambda i: (i, 0)),
                  pl.BlockSpec((F, H), lambda i: (0, 0))],
        out_specs=pl.BlockSpec((1000, H), lambda i: (i, 0)),
        out_shape=jax.ShapeDtypeStruct((N, H), bf16),
    )(x_bf, W0b)

    def p1_body(*refs):
        adj_r = refs[0:S]
        b1_r = refs[S]
        x_r = refs[S + 1:2 * S + 1]
        w1x_r, w1h_r, b0_r = refs[2 * S + 1:2 * S + 4]
        adjc_r = refs[2 * S + 4:3 * S + 4]
        h0_r = refs[3 * S + 4:4 * S + 4]
        b2_r = refs[4 * S + 4:5 * S + 4]
        for s in range(S):
            t = adj_r[s][...].astype(bf16)
            adjc_r[s][...] = t
            h0 = jnp.maximum(_dot(t, b1_r[...]) + b0_r[...], 0.0)
            h0b = h0.astype(bf16)
            h0_r[s][...] = h0b
            b2 = _dot(x_r[s][...], w1x_r[...]) + _dot(h0b, w1h_r[...])
            b2_r[s][...] = b2.astype(bf16)

    p1_out = pl.pallas_call(
        p1_body,
        grid=(nb1,),
        in_specs=(stream_spec(BM1, N, nb1)
                  + [one((N, H))]
                  + stream_spec(BM1, F, nb1)
                  + [one((F, H)), one((H, H)), one((1, H))]),
        out_specs=(local_spec(BM1, N) + local_spec(BM1, H) + local_spec(BM1, H)),
        out_shape=(shapes(R, N, bf16) + shapes(R, H, bf16) + shapes(R, H, bf16)),
        compiler_params=cparams,
    )(*(([adj] * S) + [B1] + [x_bf] * S + [W1x, W1h, b0r]))
    adjc = p1_out[0:S]
    h0 = p1_out[S:2 * S]
    B2 = p1_out[2 * S:3 * S]

    def _ksplit_dot(t, b_refs):
        acc = _dot(t[:, 0:R], b_refs[0][...])
        for u in range(1, S):
            acc += _dot(t[:, u * R:(u + 1) * R], b_refs[u][...])
        return acc

    def p2_body(*refs):
        adjc_r = refs[0:S]
        b2_r = refs[S:2 * S]
        x_r = refs[2 * S:3 * S]
        h0_r = refs[3 * S:4 * S]
        wcx_r, wch0_r, wch1_r, b1_r = refs[4 * S:4 * S + 4]
        b3_r = refs[4 * S + 4:5 * S + 4]
        for s in range(S):
            t = adjc_r[s][...]
            h1 = jnp.maximum(_ksplit_dot(t, b2_r) + b1_r[...], 0.0)
            b3 = (_dot(x_r[s][...], wcx_r[...])
                  + _dot(h0_r[s][...], wch0_r[...])
                  + _dot(h1.astype(bf16), wch1_r[...]))
            b3_r[s][...] = b3.astype(bf16)

    B3 = pl.pallas_call(
        p2_body,
        grid=(nb2,),
        in_specs=(local_spec(BM2, N)
                  + full_spec(R, H)
                  + stream_spec(BM2, F, nb2)
                  + local_spec(BM2, H)
                  + [one((F, H + C)), one((H, H + C)), one((H, H + C)), one((1, H))]),
        out_specs=local_spec(BM2, H + C),
        out_shape=shapes(R, H + C, bf16),
        compiler_params=cparams,
    )(*(list(adjc) + list(B2) + [x_bf] * S + list(h0)
        + [Wc_x, Wc_h0, Wc_h1, b1r]))

    def p3_body(*refs):
        adjc_r = refs[0:S]
        b3_r = refs[S:2 * S]
        wo2_r, b2_r, bout_r = refs[2 * S:2 * S + 3]
        b4_r = refs[2 * S + 3:3 * S + 3]
        part_r = refs[3 * S + 3:4 * S + 3]
        for s in range(S):
            acc = _ksplit_dot(adjc_r[s][...], b3_r)
            h2 = jnp.maximum(acc[:, :H] + b2_r[...], 0.0)
            part_r[s][...] = acc[:, H:] + bout_r[...]
            b4_r[s][...] = _dot(h2.astype(bf16), wo2_r[...]).astype(bf16)

    p3_out = pl.pallas_call(
        p3_body,
        grid=(nb2,),
        in_specs=(local_spec(BM2, N)
                  + full_spec(R, H + C)
                  + [one((H, C)), one((1, H)), one((1, C))]),
        out_specs=(local_spec(BM2, C) + local_spec(BM2, C)),
        out_shape=(shapes(R, C, bf16) + shapes(R, C, f32)),
        compiler_params=cparams,
    )(*(list(adjc) + list(B3) + [Wo2, b2r, boutr]))
    B4 = p3_out[0:S]
    partial = p3_out[S:2 * S]

    def p4_body(*refs):
        adjc_r = refs[0:S]
        b4_r = refs[S:2 * S]
        part_r = refs[2 * S:3 * S]
        out_r = refs[3 * S:4 * S]
        for s in range(S):
            out_r[s][...] = _ksplit_dot(adjc_r[s][...], b4_r) + part_r[s][...]

    outs = pl.pallas_call(
        p4_body,
        grid=(nb2,),
        in_specs=(local_spec(BM2, N) + full_spec(R, C) + local_spec(BM2, C)),
        out_specs=local_spec(BM2, C),
        out_shape=shapes(R, C, f32),
        compiler_params=cparams,
    )(*(list(adjc) + list(B4) + list(partial)))

    return jnp.concatenate(outs, axis=0)


# speedup vs baseline: 1.1247x; 1.1247x over previous
import jax
import jax.numpy as jnp
from jax.experimental import pallas as pl
from jax.experimental.pallas import tpu as pltpu

f32 = jnp.float32
bf16 = jnp.bfloat16


def _dot(a, b):
    return jax.lax.dot_general(a, b, (((1,), (0,)), ((), ())),
                               preferred_element_type=f32)


def kernel(x, adj, adj_high, W0, b0, W1, b1, W2, b2, W_out, b_out):
    N, F = x.shape
    H = W0.shape[1]
    C = W_out.shape[1]
    BM1 = 80
    BM2 = 400
    NBUF1, NBUF1O, NBUF2 = 8, 2, 4
    nb1 = N // BM1
    nb2 = N // BM2

    x_bf = x.astype(bf16)
    W0b = W0.astype(bf16)
    W1x = W1[:F].astype(bf16)
    W1h = W1[F:].astype(bf16)
    Wc_x = jnp.concatenate([W2[:F], W_out[:F]], axis=1).astype(bf16)
    Wc_h0 = jnp.concatenate([W2[F:F + H], W_out[F:F + H]], axis=1).astype(bf16)
    Wc_h1 = jnp.concatenate([W2[F + H:], W_out[F + H:F + 2 * H]], axis=1).astype(bf16)
    Wo2 = W_out[F + 2 * H:].astype(bf16)
    b0r = b0.reshape(1, H).astype(f32)
    b1r = b1.reshape(1, H).astype(f32)
    b2r = b2.reshape(1, H).astype(f32)
    boutr = b_out.reshape(1, C).astype(f32)

    def big_spec(bm, nbuf):
        return pl.BlockSpec((bm, N), lambda i: (i, 0),
                            pipeline_mode=pl.Buffered(buffer_count=nbuf))

    def mega_body(adj_hbm, x_v, w0_v, w1x_v, w1h_v, wcx_v, wch0_v, wch1_v,
                  wo2_v, b0_v, b1_v, b2_v, bout_v,
                  out_v, adjc_hbm,
                  B1s, B2s, B3s, B4s, h0s, parts, cnt):
        B1s[...] = _dot(x_v[...], w0_v[...]).astype(bf16)

        def p1_body(adj_blk, adjc_blk):
            i = cnt[0]
            r = pl.ds(i * BM1, BM1)
            t = adj_blk[...].astype(bf16)
            adjc_blk[...] = t
            h0 = jnp.maximum(_dot(t, B1s[...]) + b0_v[...], 0.0)
            h0b = h0.astype(bf16)
            h0s[r, :] = h0b
            B2s[r, :] = (_dot(x_v[r, :], w1x_v[...])
                         + _dot(h0b, w1h_v[...])).astype(bf16)
            cnt[0] = i + 1

        cnt[0] = 0
        pltpu.emit_pipeline(
            p1_body, grid=(nb1,),
            in_specs=[big_spec(BM1, NBUF1)],
            out_specs=[big_spec(BM1, NBUF1O)],
        )(adj_hbm, adjc_hbm)

        def p2_body(adjc_blk):
            i = cnt[1]
            r = pl.ds(i * BM2, BM2)
            h1 = jnp.maximum(_dot(adjc_blk[...], B2s[...]) + b1_v[...], 0.0)
            b3 = (_dot(x_v[r, :], wcx_v[...])
                  + _dot(h0s[r, :], wch0_v[...])
                  + _dot(h1.astype(bf16), wch1_v[...]))
            B3s[r, :] = b3.astype(bf16)
            cnt[1] = i + 1

        cnt[1] = 0
        pltpu.emit_pipeline(
            p2_body, grid=(nb2,),
            in_specs=[big_spec(BM2, NBUF2)],
        )(adjc_hbm)

        def p3_body(adjc_blk):
            i = cnt[2]
            r = pl.ds(i * BM2, BM2)
            acc = _dot(adjc_blk[...], B3s[...])
            h2 = jnp.maximum(acc[:, :H] + b2_v[...], 0.0)
            parts[r, :] = acc[:, H:] + bout_v[...]
            B4s[r, :] = _dot(h2.astype(bf16), wo2_v[...]).astype(bf16)
            cnt[2] = i + 1

        cnt[2] = 0
        pltpu.emit_pipeline(
            p3_body, grid=(nb2,),
            in_specs=[big_spec(BM2, NBUF2)],
        )(adjc_hbm)

        def p4_body(adjc_blk):
            i = cnt[3]
            r = pl.ds(i * BM2, BM2)
            out_v[r, :] = _dot(adjc_blk[...], B4s[...]) + parts[r, :]
            cnt[3] = i + 1

        cnt[3] = 0
        pltpu.emit_pipeline(
            p4_body, grid=(nb2,),
            in_specs=[big_spec(BM2, NBUF2)],
        )(adjc_hbm)

    vmem = pl.BlockSpec(memory_space=pltpu.MemorySpace.VMEM)
    hbm = pl.BlockSpec(memory_space=pltpu.MemorySpace.HBM)

    out, _ = pl.pallas_call(
        mega_body,
        in_specs=[hbm] + [vmem] * 12,
        out_specs=[vmem, hbm],
        out_shape=[jax.ShapeDtypeStruct((N, C), f32),
                   jax.ShapeDtypeStruct((N, N), bf16)],
        scratch_shapes=[pltpu.VMEM((N, H), bf16),
                        pltpu.VMEM((N, H), bf16),
                        pltpu.VMEM((N, H + C), bf16),
                        pltpu.VMEM((N, C), bf16),
                        pltpu.VMEM((N, H), bf16),
                        pltpu.VMEM((N, C), f32),
                        pltpu.SMEM((4,), jnp.int32)],
    )(adj, x_bf, W0b, W1x, W1h, Wc_x, Wc_h0, Wc_h1, Wo2,
      b0r, b1r, b2r, boutr)

    return out
